# baseline (speedup 1.0000x reference)
import jax
import jax.numpy as jnp
from jax import lax
from jax.experimental import pallas as pl
from jax.experimental.pallas import tpu as pltpu

_Q, _N, _F = 1024, 20000, 64
_NB = 2048
_NBLK = 10
_BIGF = 3.0e38


def _side_step(a_ref, p_ref, nb, minv, idxf):
    a = a_ref[...]
    p = p_ref[...]
    a2 = (a[:, 0:1] * a[:, 0:1] + a[:, 1:2] * a[:, 1:2]) + a[:, 2:3] * a[:, 2:3]
    pp = p * p
    eye8 = jnp.where(
        lax.broadcasted_iota(jnp.int32, (8, 3), 0)
        == lax.broadcasted_iota(jnp.int32, (8, 3), 1), 1.0, 0.0)
    rows = lax.dot_general(eye8, pp, (((1,), (1,)), ((), ())),
                           precision=lax.Precision.HIGHEST,
                           preferred_element_type=jnp.float32)
    b2 = (rows[0:1] + rows[1:2]) + rows[2:3]
    mm2 = lax.dot_general(a, p * (-2.0), (((1,), (1,)), ((), ())),
                          preferred_element_type=jnp.float32)
    d = (a2 + b2) + mm2
    colf = (lax.broadcasted_iota(jnp.int32, (1, _NB), 1)
            + nb * _NB).astype(jnp.float32)
    colb = jnp.broadcast_to(colf, d.shape)
    d = jnp.where(colb < float(_N), d, _BIGF)
    bmin = jnp.min(d, axis=1, keepdims=True)
    bidx = jnp.min(jnp.where(d == bmin, colb, _BIGF),
                   axis=1, keepdims=True)

    @pl.when(nb == 0)
    def _():
        minv[...] = jnp.full_like(minv, _BIGF)
        idxf[...] = jnp.zeros_like(idxf)

    better = bmin < minv[...]
    minv[...] = jnp.where(better, bmin, minv[...])
    idxf[...] = jnp.where(better, bidx, idxf[...])


def _fused_body(ll_ref, lr_ref, pl_ref, pr_ref, fl_any, fr_any,
                w1_ref, b1_ref, w2_ref, b2_ref, out_ref,
                minvl, idxfl, minvr, idxfr, idx32, idx_smem,
                flv, frv, xv, sem_l, sem_r):
    nb = pl.program_id(0)

    @pl.when(nb == 0)
    def _():
        pltpu.make_async_copy(fl_any, flv, sem_l).start()
        pltpu.make_async_copy(fr_any, frv, sem_r).start()

    _side_step(ll_ref, pl_ref, nb, minvl, idxfl)
    _side_step(lr_ref, pr_ref, nb, minvr, idxfr)

    @pl.when(nb == _NBLK - 1)
    def _():
        idx32[...] = jnp.swapaxes(idxfl[...].astype(jnp.int32), 0, 1)
        pltpu.sync_copy(idx32, idx_smem.at[0])
        idx32[...] = jnp.swapaxes(idxfr[...].astype(jnp.int32), 0, 1)
        pltpu.sync_copy(idx32, idx_smem.at[1])
        pltpu.make_async_copy(fl_any, flv, sem_l).wait()
        pltpu.make_async_copy(fr_any, frv, sem_r).wait()

        def gather_one(i, carry):
            rl = idx_smem[0, 0, i]
            rr = idx_smem[1, 0, i]
            xv[pl.ds(i, 1), 0:_F] = flv[pl.ds(rl, 1), :]
            xv[pl.ds(i, 1), _F:2 * _F] = frv[pl.ds(rr, 1), :]
            return carry

        lax.fori_loop(0, _Q, gather_one, 0)
        x = xv[...]
        h = jnp.maximum(
            jnp.dot(x, w1_ref[...], preferred_element_type=jnp.float32)
            + b1_ref[...], 0.0)
        out_ref[...] = (jnp.dot(h, w2_ref[...],
                                preferred_element_type=jnp.float32)
                        + b2_ref[...])


def kernel(locs_left, locs_right, pos_left, pos_right,
           feats_left, feats_right, W1, b1, W2, b2):
    out = pl.pallas_call(
        _fused_body,
        grid=(_NBLK,),
        in_specs=[
            pl.BlockSpec((_Q, 3), lambda nb: (0, 0)),
            pl.BlockSpec((_Q, 3), lambda nb: (0, 0)),
            pl.BlockSpec((_NB, 3), lambda nb: (nb, 0)),
            pl.BlockSpec((_NB, 3), lambda nb: (nb, 0)),
            pl.BlockSpec(memory_space=pl.ANY),
            pl.BlockSpec(memory_space=pl.ANY),
            pl.BlockSpec((2 * _F, 2 * _F), lambda nb: (0, 0)),
            pl.BlockSpec((1, 2 * _F), lambda nb: (0, 0)),
            pl.BlockSpec((2 * _F, 1), lambda nb: (0, 0)),
            pl.BlockSpec((1, 1), lambda nb: (0, 0)),
        ],
        out_specs=pl.BlockSpec((_Q, 1), lambda nb: (0, 0)),
        out_shape=jax.ShapeDtypeStruct((_Q, 1), jnp.float32),
        scratch_shapes=[
            pltpu.VMEM((_Q, 1), jnp.float32),
            pltpu.VMEM((_Q, 1), jnp.float32),
            pltpu.VMEM((_Q, 1), jnp.float32),
            pltpu.VMEM((_Q, 1), jnp.float32),
            pltpu.VMEM((1, _Q), jnp.int32),
            pltpu.SMEM((2, 1, _Q), jnp.int32),
            pltpu.VMEM((_N, _F), jnp.float32),
            pltpu.VMEM((_N, _F), jnp.float32),
            pltpu.VMEM((_Q, 2 * _F), jnp.float32),
            pltpu.SemaphoreType.DMA,
            pltpu.SemaphoreType.DMA,
        ],
        compiler_params=pltpu.CompilerParams(
            dimension_semantics=("arbitrary",),
            vmem_limit_bytes=100 * 1024 * 1024),
    )(locs_left, locs_right, pos_left, pos_right, feats_left, feats_right,
      W1, b1.reshape(1, -1), W2, b2.reshape(1, 1))
    return out.reshape(-1)


# speedup vs baseline: 1.4518x; 1.4518x over previous
import jax
import jax.numpy as jnp
from jax import lax
from jax.experimental import pallas as pl
from jax.experimental.pallas import tpu as pltpu

_Q, _N, _F = 1024, 20000, 64
_NB = 2048
_NBLK = 10
_BIGF = 3.0e38


def _side_step(a_ref, p_ref, nb, minv, idxf):
    a = a_ref[...]
    p = p_ref[...]
    a2 = (a[:, 0:1] * a[:, 0:1] + a[:, 1:2] * a[:, 1:2]) + a[:, 2:3] * a[:, 2:3]
    pp = p * p
    eye8 = jnp.where(
        lax.broadcasted_iota(jnp.int32, (8, 3), 0)
        == lax.broadcasted_iota(jnp.int32, (8, 3), 1), 1.0, 0.0)
    rows = lax.dot_general(eye8, pp, (((1,), (1,)), ((), ())),
                           precision=lax.Precision.HIGHEST,
                           preferred_element_type=jnp.float32)
    b2 = (rows[0:1] + rows[1:2]) + rows[2:3]
    mm2 = lax.dot_general(a, p * (-2.0), (((1,), (1,)), ((), ())),
                          preferred_element_type=jnp.float32)
    d = (a2 + b2) + mm2
    colf = (lax.broadcasted_iota(jnp.int32, (1, _NB), 1)
            + nb * _NB).astype(jnp.float32)
    colb = jnp.broadcast_to(colf, d.shape)
    d = jnp.where(colb < float(_N), d, _BIGF)
    bmin = jnp.min(d, axis=1, keepdims=True)
    bidx = jnp.min(jnp.where(d == bmin, colb, _BIGF),
                   axis=1, keepdims=True)

    @pl.when(nb == 0)
    def _():
        minv[...] = jnp.full_like(minv, _BIGF)
        idxf[...] = jnp.zeros_like(idxf)

    better = bmin < minv[...]
    minv[...] = jnp.where(better, bmin, minv[...])
    idxf[...] = jnp.where(better, bidx, idxf[...])


def _fused_body(ll_ref, lr_ref, pl_ref, pr_ref, fl_any, fr_any,
                w1_ref, b1_ref, w2_ref, b2_ref, out_ref,
                minvl, idxfl, minvr, idxfr, idx32, idx_smem,
                flv, frv, xv, sem_l, sem_r):
    nb = pl.program_id(0)

    @pl.when(nb == 0)
    def _():
        pltpu.make_async_copy(fl_any, flv, sem_l).start()
        pltpu.make_async_copy(fr_any, frv, sem_r).start()

    _side_step(ll_ref, pl_ref, nb, minvl, idxfl)
    _side_step(lr_ref, pr_ref, nb, minvr, idxfr)

    @pl.when(nb == _NBLK - 1)
    def _():
        idx32[...] = jnp.swapaxes(idxfl[...].astype(jnp.int32), 0, 1)
        pltpu.sync_copy(idx32, idx_smem.at[0])
        idx32[...] = jnp.swapaxes(idxfr[...].astype(jnp.int32), 0, 1)
        pltpu.sync_copy(idx32, idx_smem.at[1])
        pltpu.make_async_copy(fl_any, flv, sem_l).wait()
        pltpu.make_async_copy(fr_any, frv, sem_r).wait()

        def gather_one(i, carry):
            rl = idx_smem[0, 0, i]
            rr = idx_smem[1, 0, i]
            xv[pl.ds(i, 1), 0:_F] = flv[pl.ds(rl, 1), :]
            xv[pl.ds(i, 1), _F:2 * _F] = frv[pl.ds(rr, 1), :]
            return carry

        lax.fori_loop(0, _Q, gather_one, 0, unroll=8)
        x = xv[...]
        h = jnp.maximum(
            jnp.dot(x, w1_ref[...], preferred_element_type=jnp.float32)
            + b1_ref[...], 0.0)
        out_ref[...] = (jnp.dot(h, w2_ref[...],
                                preferred_element_type=jnp.float32)
                        + b2_ref[...])


def kernel(locs_left, locs_right, pos_left, pos_right,
           feats_left, feats_right, W1, b1, W2, b2):
    out = pl.pallas_call(
        _fused_body,
        grid=(_NBLK,),
        in_specs=[
            pl.BlockSpec((_Q, 3), lambda nb: (0, 0)),
            pl.BlockSpec((_Q, 3), lambda nb: (0, 0)),
            pl.BlockSpec((_NB, 3), lambda nb: (nb, 0)),
            pl.BlockSpec((_NB, 3), lambda nb: (nb, 0)),
            pl.BlockSpec(memory_space=pl.ANY),
            pl.BlockSpec(memory_space=pl.ANY),
            pl.BlockSpec((2 * _F, 2 * _F), lambda nb: (0, 0)),
            pl.BlockSpec((1, 2 * _F), lambda nb: (0, 0)),
            pl.BlockSpec((2 * _F, 1), lambda nb: (0, 0)),
            pl.BlockSpec((1, 1), lambda nb: (0, 0)),
        ],
        out_specs=pl.BlockSpec((_Q, 1), lambda nb: (0, 0)),
        out_shape=jax.ShapeDtypeStruct((_Q, 1), jnp.float32),
        scratch_shapes=[
            pltpu.VMEM((_Q, 1), jnp.float32),
            pltpu.VMEM((_Q, 1), jnp.float32),
            pltpu.VMEM((_Q, 1), jnp.float32),
            pltpu.VMEM((_Q, 1), jnp.float32),
            pltpu.VMEM((1, _Q), jnp.int32),
            pltpu.SMEM((2, 1, _Q), jnp.int32),
            pltpu.VMEM((_N, _F), jnp.float32),
            pltpu.VMEM((_N, _F), jnp.float32),
            pltpu.VMEM((_Q, 2 * _F), jnp.float32),
            pltpu.SemaphoreType.DMA,
            pltpu.SemaphoreType.DMA,
        ],
        compiler_params=pltpu.CompilerParams(
            dimension_semantics=("arbitrary",),
            vmem_limit_bytes=100 * 1024 * 1024),
    )(locs_left, locs_right, pos_left, pos_right, feats_left, feats_right,
      W1, b1.reshape(1, -1), W2, b2.reshape(1, 1))
    return out.reshape(-1)


# speedup vs baseline: 1.5105x; 1.0404x over previous
import jax
import jax.numpy as jnp
from jax import lax
from jax.experimental import pallas as pl
from jax.experimental.pallas import tpu as pltpu

_Q, _N, _F = 1024, 20000, 64
_NB = 2048
_NBLK = 10
_BIGF = 3.0e38


def _side_step(a_ref, p_ref, nb, minv, idxf):
    a = a_ref[...]
    p = p_ref[...]
    a2 = (a[:, 0:1] * a[:, 0:1] + a[:, 1:2] * a[:, 1:2]) + a[:, 2:3] * a[:, 2:3]
    pp = p * p
    eye8 = jnp.where(
        lax.broadcasted_iota(jnp.int32, (8, 3), 0)
        == lax.broadcasted_iota(jnp.int32, (8, 3), 1), 1.0, 0.0)
    rows = lax.dot_general(eye8, pp, (((1,), (1,)), ((), ())),
                           precision=lax.Precision.HIGHEST,
                           preferred_element_type=jnp.float32)
    b2 = (rows[0:1] + rows[1:2]) + rows[2:3]
    mm2 = lax.dot_general(a, p * (-2.0), (((1,), (1,)), ((), ())),
                          preferred_element_type=jnp.float32)
    d = (a2 + b2) + mm2
    colf = (lax.broadcasted_iota(jnp.int32, (1, _NB), 1)
            + nb * _NB).astype(jnp.float32)
    colb = jnp.broadcast_to(colf, d.shape)
    d = jnp.where(colb < float(_N), d, _BIGF)
    bmin = jnp.min(d, axis=1, keepdims=True)
    bidx = jnp.min(jnp.where(d == bmin, colb, _BIGF),
                   axis=1, keepdims=True)

    @pl.when(nb == 0)
    def _():
        minv[...] = jnp.full_like(minv, _BIGF)
        idxf[...] = jnp.zeros_like(idxf)

    better = bmin < minv[...]
    minv[...] = jnp.where(better, bmin, minv[...])
    idxf[...] = jnp.where(better, bidx, idxf[...])


def _fused_body(ll_ref, lr_ref, pl_ref, pr_ref, fl_any, fr_any,
                w1_ref, b1_ref, w2_ref, b2_ref, out_ref,
                minvl, idxfl, minvr, idxfr, idx32, idx_smem,
                flv, frv, xv, sem_l, sem_r):
    nb = pl.program_id(0)

    @pl.when(nb == 0)
    def _():
        pltpu.make_async_copy(fl_any, flv, sem_l).start()
        pltpu.make_async_copy(fr_any, frv, sem_r).start()

    _side_step(ll_ref, pl_ref, nb, minvl, idxfl)
    _side_step(lr_ref, pr_ref, nb, minvr, idxfr)

    @pl.when(nb == _NBLK - 1)
    def _():
        idx32[...] = jnp.swapaxes(idxfl[...].astype(jnp.int32), 0, 1)
        pltpu.sync_copy(idx32, idx_smem.at[0])
        idx32[...] = jnp.swapaxes(idxfr[...].astype(jnp.int32), 0, 1)
        pltpu.sync_copy(idx32, idx_smem.at[1])
        pltpu.make_async_copy(fl_any, flv, sem_l).wait()
        pltpu.make_async_copy(fr_any, frv, sem_r).wait()

        def gather_one(i, carry):
            rl = idx_smem[0, 0, i]
            rr = idx_smem[1, 0, i]
            xv[pl.ds(i, 1), 0:_F] = flv[pl.ds(rl, 1), :]
            xv[pl.ds(i, 1), _F:2 * _F] = frv[pl.ds(rr, 1), :]
            return carry

        lax.fori_loop(0, _Q, gather_one, 0, unroll=32)
        x = xv[...]
        h = jnp.maximum(
            jnp.dot(x, w1_ref[...], preferred_element_type=jnp.float32)
            + b1_ref[...], 0.0)
        out_ref[...] = (jnp.dot(h, w2_ref[...],
                                preferred_element_type=jnp.float32)
                        + b2_ref[...])


def kernel(locs_left, locs_right, pos_left, pos_right,
           feats_left, feats_right, W1, b1, W2, b2):
    out = pl.pallas_call(
        _fused_body,
        grid=(_NBLK,),
        in_specs=[
            pl.BlockSpec((_Q, 3), lambda nb: (0, 0)),
            pl.BlockSpec((_Q, 3), lambda nb: (0, 0)),
            pl.BlockSpec((_NB, 3), lambda nb: (nb, 0)),
            pl.BlockSpec((_NB, 3), lambda nb: (nb, 0)),
            pl.BlockSpec(memory_space=pl.ANY),
            pl.BlockSpec(memory_space=pl.ANY),
            pl.BlockSpec((2 * _F, 2 * _F), lambda nb: (0, 0)),
            pl.BlockSpec((1, 2 * _F), lambda nb: (0, 0)),
            pl.BlockSpec((2 * _F, 1), lambda nb: (0, 0)),
            pl.BlockSpec((1, 1), lambda nb: (0, 0)),
        ],
        out_specs=pl.BlockSpec((_Q, 1), lambda nb: (0, 0)),
        out_shape=jax.ShapeDtypeStruct((_Q, 1), jnp.float32),
        scratch_shapes=[
            pltpu.VMEM((_Q, 1), jnp.float32),
            pltpu.VMEM((_Q, 1), jnp.float32),
            pltpu.VMEM((_Q, 1), jnp.float32),
            pltpu.VMEM((_Q, 1), jnp.float32),
            pltpu.VMEM((1, _Q), jnp.int32),
            pltpu.SMEM((2, 1, _Q), jnp.int32),
            pltpu.VMEM((_N, _F), jnp.float32),
            pltpu.VMEM((_N, _F), jnp.float32),
            pltpu.VMEM((_Q, 2 * _F), jnp.float32),
            pltpu.SemaphoreType.DMA,
            pltpu.SemaphoreType.DMA,
        ],
        compiler_params=pltpu.CompilerParams(
            dimension_semantics=("arbitrary",),
            vmem_limit_bytes=100 * 1024 * 1024),
    )(locs_left, locs_right, pos_left, pos_right, feats_left, feats_right,
      W1, b1.reshape(1, -1), W2, b2.reshape(1, 1))
    return out.reshape(-1)


# speedup vs baseline: 2.2527x; 1.4914x over previous
import jax
import jax.numpy as jnp
from jax import lax
from jax.experimental import pallas as pl
from jax.experimental.pallas import tpu as pltpu

_Q, _N, _F = 1024, 20000, 64
_NB = 2048
_NBLK = 10
_BIGF = 3.0e38


def _side_step(a_ref, pt_ref, nb, minv, idxf):
    a = a_ref[...]
    pt = pt_ref[...]
    a2 = (a[:, 0:1] * a[:, 0:1] + a[:, 1:2] * a[:, 1:2]) + a[:, 2:3] * a[:, 2:3]
    b2 = (pt[0:1] * pt[0:1] + pt[1:2] * pt[1:2]) + pt[2:3] * pt[2:3]
    mm2 = lax.dot_general(a, pt * (-2.0), (((1,), (0,)), ((), ())),
                          preferred_element_type=jnp.float32)
    d = (a2 + b2) + mm2
    colf = (lax.broadcasted_iota(jnp.int32, (1, _NB), 1)
            + nb * _NB).astype(jnp.float32)
    colb = jnp.broadcast_to(colf, d.shape)
    bmin = jnp.min(d, axis=1, keepdims=True)
    bidx = jnp.min(jnp.where(d == bmin, colb, _BIGF),
                   axis=1, keepdims=True)

    @pl.when(nb == 0)
    def _():
        minv[...] = jnp.full_like(minv, _BIGF)
        idxf[...] = jnp.zeros_like(idxf)

    better = bmin < minv[...]
    minv[...] = jnp.where(better, bmin, minv[...])
    idxf[...] = jnp.where(better, bidx, idxf[...])


def _fused_body(ll_ref, lr_ref, pl_ref, pr_ref, fl_any, fr_any,
                w1_ref, b1_ref, w2_ref, b2_ref, out_ref,
                minvl, idxfl, minvr, idxfr, idx32, idx_smem,
                flv, frv, xv, sem_l, sem_r):
    nb = pl.program_id(0)

    @pl.when(nb == 0)
    def _():
        pltpu.make_async_copy(fl_any, flv, sem_l).start()
        pltpu.make_async_copy(fr_any, frv, sem_r).start()

    _side_step(ll_ref, pl_ref, nb, minvl, idxfl)
    _side_step(lr_ref, pr_ref, nb, minvr, idxfr)

    @pl.when(nb == _NBLK - 1)
    def _():
        idx32[...] = jnp.swapaxes(idxfl[...].astype(jnp.int32), 0, 1)
        pltpu.sync_copy(idx32, idx_smem.at[0])
        idx32[...] = jnp.swapaxes(idxfr[...].astype(jnp.int32), 0, 1)
        pltpu.sync_copy(idx32, idx_smem.at[1])
        pltpu.make_async_copy(fl_any, flv, sem_l).wait()
        pltpu.make_async_copy(fr_any, frv, sem_r).wait()

        def gather_one(i, carry):
            rl = idx_smem[0, 0, i]
            rr = idx_smem[1, 0, i]
            xv[pl.ds(i, 1), 0:_F] = flv[pl.ds(rl, 1), :]
            xv[pl.ds(i, 1), _F:2 * _F] = frv[pl.ds(rr, 1), :]
            return carry

        lax.fori_loop(0, _Q, gather_one, 0, unroll=32)
        x = xv[...]
        h = jnp.maximum(
            jnp.dot(x, w1_ref[...], preferred_element_type=jnp.float32)
            + b1_ref[...], 0.0)
        out_ref[...] = (jnp.dot(h, w2_ref[...],
                                preferred_element_type=jnp.float32)
                        + b2_ref[...])


def kernel(locs_left, locs_right, pos_left, pos_right,
           feats_left, feats_right, W1, b1, W2, b2):
    ptl = jnp.pad(pos_left.T, ((0, 0), (0, _NBLK * _NB - _N)),
                  constant_values=1e15)
    ptr = jnp.pad(pos_right.T, ((0, 0), (0, _NBLK * _NB - _N)),
                  constant_values=1e15)
    out = pl.pallas_call(
        _fused_body,
        grid=(_NBLK,),
        in_specs=[
            pl.BlockSpec((_Q, 3), lambda nb: (0, 0)),
            pl.BlockSpec((_Q, 3), lambda nb: (0, 0)),
            pl.BlockSpec((3, _NB), lambda nb: (0, nb)),
            pl.BlockSpec((3, _NB), lambda nb: (0, nb)),
            pl.BlockSpec(memory_space=pl.ANY),
            pl.BlockSpec(memory_space=pl.ANY),
            pl.BlockSpec((2 * _F, 2 * _F), lambda nb: (0, 0)),
            pl.BlockSpec((1, 2 * _F), lambda nb: (0, 0)),
            pl.BlockSpec((2 * _F, 1), lambda nb: (0, 0)),
            pl.BlockSpec((1, 1), lambda nb: (0, 0)),
        ],
        out_specs=pl.BlockSpec((_Q, 1), lambda nb: (0, 0)),
        out_shape=jax.ShapeDtypeStruct((_Q, 1), jnp.float32),
        scratch_shapes=[
            pltpu.VMEM((_Q, 1), jnp.float32),
            pltpu.VMEM((_Q, 1), jnp.float32),
            pltpu.VMEM((_Q, 1), jnp.float32),
            pltpu.VMEM((_Q, 1), jnp.float32),
            pltpu.VMEM((1, _Q), jnp.int32),
            pltpu.SMEM((2, 1, _Q), jnp.int32),
            pltpu.VMEM((_N, _F), jnp.float32),
            pltpu.VMEM((_N, _F), jnp.float32),
            pltpu.VMEM((_Q, 2 * _F), jnp.float32),
            pltpu.SemaphoreType.DMA,
            pltpu.SemaphoreType.DMA,
        ],
        compiler_params=pltpu.CompilerParams(
            dimension_semantics=("arbitrary",),
            vmem_limit_bytes=100 * 1024 * 1024),
    )(locs_left, locs_right, ptl, ptr, feats_left, feats_right,
      W1, b1.reshape(1, -1), W2, b2.reshape(1, 1))
    return out.reshape(-1)
